# baseline (speedup 1.0000x reference)
import jax
import jax.numpy as jnp
from jax import lax
from jax.experimental import pallas as pl
from jax.experimental.pallas import tpu as pltpu
from jax.experimental.pallas import tpu_sc as plsc

PAD = 1
B = 4
S = 8192
D = 64
NCORES = 2
NSUB = 16
NW = NCORES * NSUB
CHUNK = (B * S) // NW
CPR = S // CHUNK
ROWS_PER_CORE = B // NCORES
NVREG = CHUNK // 16
IDX_W = 128
NGATHER = CHUNK // IDX_W


def _body(tok_hbm, tab_hbm, out_hbm, tok_v, pos_v, aloc_v, mask_v, rows_v,
          trans_v, tot_v, grp_v, shared_tot, sem):
    c = lax.axis_index("c")
    s = lax.axis_index("s")
    row = c * ROWS_PER_CORE + s // CPR
    cir = s % CPR
    off = cir * CHUNK

    pltpu.sync_copy(tok_hbm.at[pl.ds(cir * (CHUNK // 128), CHUNK // 128),
                               pl.ds(row, 1), :], tok_v)

    pad_v = jnp.broadcast_to(jnp.int32(PAD), (16,))

    run = jnp.int32(0)
    for i in range(NVREG):
        t = tok_v[i // 8, 0, pl.ds((i % 8) * 16, 16)]
        m = (t != pad_v).astype(jnp.int32)
        cum = plsc.cumsum(m) + jnp.broadcast_to(run, (16,))
        aloc_v[pl.ds(i * 16, 16)] = cum * m
        mask_v[pl.ds(i * 16, 16)] = m
        run = jnp.max(cum)
    tot_v[...] = jnp.broadcast_to(run, (16,))
    pltpu.sync_copy(tot_v, shared_tot.at[s])
    plsc.subcore_barrier()

    grp_base = (s // CPR) * CPR
    pltpu.sync_copy(shared_tot.at[pl.ds(grp_base, CPR)], grp_v)
    offset = jnp.int32(0)
    for j in range(CPR):
        tj = jnp.max(grp_v[j, :])
        offset = offset + jnp.where(j < cir, tj, jnp.int32(0))

    off_v = jnp.broadcast_to(offset, (16,))
    for i in range(NVREG):
        a = aloc_v[pl.ds(i * 16, 16)]
        m = mask_v[pl.ds(i * 16, 16)]
        pos = a + off_v * m + pad_v
        pos_v[i // (IDX_W // 16), pl.ds((i % (IDX_W // 16)) * 16, 16)] = pos

    handles = [
        pltpu.async_copy(tab_hbm.at[pos_v.at[j]],
                         rows_v.at[pl.ds(j * IDX_W, IDX_W)], sem)
        for j in range(NGATHER)
    ]
    for h in handles:
        h.wait()

    row_iota = lax.iota(jnp.int32, 16)

    st0 = cir * (CHUNK // 128)
    nst = CHUNK // 128

    def q_body(q, carry):
        p = q % 2
        @pl.when(q >= 2)
        def _():
            for _k in range(16):
                pltpu.make_async_copy(
                    trans_v.at[pl.ds(0, 8), pl.ds(0, 128)],
                    out_hbm.at[row, 0, st0, :, :],
                    sem).wait()
        idx0_v = row_iota + jnp.broadcast_to(p * 16, (16,))
        for g in range(CHUNK // 8):
            vals = [rows_v[g * 8 + k, pl.ds(q * 16, 16)] for k in range(8)]
            for k in range(8):
                idx1_v = jnp.broadcast_to(jnp.int32(g * 8 + k), (16,))
                plsc.store_scatter(trans_v, [idx0_v, idx1_v], vals[k])
        for etp in range(2):
            for st in range(nst):
                pltpu.async_copy(
                    trans_v.at[pl.ds(p * 16 + etp * 8, 8),
                               pl.ds(st * 128, 128)],
                    out_hbm.at[row, 2 * q + etp, st0 + st, :, :],
                    sem)
        return carry

    lax.fori_loop(0, D // 16, q_body, jnp.int32(0))
    for _k in range(32):
        pltpu.make_async_copy(
            trans_v.at[pl.ds(0, 8), pl.ds(0, 128)],
            out_hbm.at[row, 0, pl.ds(st0, 8), :, :].at[0],
            sem).wait()


def kernel(tokens, embed_table):
    tok = (tokens.astype(jnp.int32)
           .reshape(B, S // 128, 128).transpose(1, 0, 2))
    mesh = plsc.VectorSubcoreMesh(core_axis_name="c", subcore_axis_name="s")
    run_k = pl.kernel(
        _body,
        mesh=mesh,
        compiler_params=pltpu.CompilerParams(
            use_tc_tiling_on_sc=False, needs_layout_passes=False),
        out_type=jax.ShapeDtypeStruct((B, D // 8, S // 128, 8, 128),
                                      jnp.float32),
        scratch_types=[
            pltpu.VMEM((CHUNK // 128, 1, 128), jnp.int32),
            pltpu.VMEM((NGATHER, IDX_W), jnp.int32),
            pltpu.VMEM((CHUNK,), jnp.int32),
            pltpu.VMEM((CHUNK,), jnp.int32),
            pltpu.VMEM((CHUNK, D), jnp.float32),
            pltpu.VMEM((32, CHUNK + 1), jnp.float32),
            pltpu.VMEM((16,), jnp.int32),
            pltpu.VMEM((CPR, 16), jnp.int32),
            pltpu.VMEM_SHARED((NSUB, 16), jnp.int32),
            pltpu.SemaphoreType.DMA,
        ],
    )
    out5 = run_k(tok, embed_table)
    return out5.transpose(0, 2, 4, 1, 3).reshape(B, S, D)


# speedup vs baseline: 1.0737x; 1.0737x over previous
import jax
import jax.numpy as jnp
from jax import lax
from jax.experimental import pallas as pl
from jax.experimental.pallas import tpu as pltpu
from jax.experimental.pallas import tpu_sc as plsc

PAD = 1
B = 4
S = 8192
D = 64
NCORES = 2
NSUB = 16
NW = NCORES * NSUB
CHUNK = (B * S) // NW
CPR = S // CHUNK
ROWS_PER_CORE = B // NCORES
NVREG = CHUNK // 16
IDX_W = 128
NGATHER = CHUNK // IDX_W


def _body(tok_hbm, tab_hbm, out_hbm, tok_v, pos_v, aloc_v, mask_v, rows_v,
          trans_v, tot_v, grp_v, shared_tot, sem):
    c = lax.axis_index("c")
    s = lax.axis_index("s")
    row = c * ROWS_PER_CORE + s // CPR
    cir = s % CPR
    off = cir * CHUNK

    pltpu.sync_copy(tok_hbm.at[pl.ds(cir * (CHUNK // 128), CHUNK // 128),
                               pl.ds(row, 1), :], tok_v)

    pad_v = jnp.broadcast_to(jnp.int32(PAD), (16,))

    run = jnp.int32(0)
    for i in range(NVREG):
        t = tok_v[i // 8, 0, pl.ds((i % 8) * 16, 16)]
        m = (t != pad_v).astype(jnp.int32)
        cum = plsc.cumsum(m) + jnp.broadcast_to(run, (16,))
        aloc_v[pl.ds(i * 16, 16)] = cum * m
        mask_v[pl.ds(i * 16, 16)] = m
        run = run + jnp.sum(m)
    tot_v[...] = jnp.broadcast_to(run, (16,))
    pltpu.sync_copy(tot_v, shared_tot.at[s])
    plsc.subcore_barrier()

    grp_base = (s // CPR) * CPR
    pltpu.sync_copy(shared_tot.at[pl.ds(grp_base, CPR)], grp_v)
    offset = jnp.int32(0)
    for j in range(CPR):
        tj = jnp.max(grp_v[j, :])
        offset = offset + jnp.where(j < cir, tj, jnp.int32(0))

    off_v = jnp.broadcast_to(offset, (16,))
    for i in range(NVREG):
        a = aloc_v[pl.ds(i * 16, 16)]
        m = mask_v[pl.ds(i * 16, 16)]
        pos = a + off_v * m + pad_v
        pos_v[i // (IDX_W // 16), pl.ds((i % (IDX_W // 16)) * 16, 16)] = pos

    handles = [
        pltpu.async_copy(tab_hbm.at[pos_v.at[j]],
                         rows_v.at[pl.ds(j * IDX_W, IDX_W)], sem)
        for j in range(NGATHER)
    ]
    for h in handles:
        h.wait()

    row_iota = lax.iota(jnp.int32, 16)

    st0 = cir * (CHUNK // 128)
    nst = CHUNK // 128

    def q_body(q, carry):
        p = q % 2
        @pl.when(q >= 2)
        def _():
            for _k in range(16):
                pltpu.make_async_copy(
                    trans_v.at[pl.ds(0, 8), pl.ds(0, 128)],
                    out_hbm.at[row, 0, st0, :, :],
                    sem).wait()
        idx0_v = row_iota + jnp.broadcast_to(p * 16, (16,))
        for g in range(CHUNK // 8):
            vals = [rows_v[g * 8 + k, pl.ds(q * 16, 16)] for k in range(8)]
            for k in range(8):
                idx1_v = jnp.broadcast_to(jnp.int32(g * 8 + k), (16,))
                plsc.store_scatter(trans_v, [idx0_v, idx1_v], vals[k])
        for etp in range(2):
            for st in range(nst):
                pltpu.async_copy(
                    trans_v.at[pl.ds(p * 16 + etp * 8, 8),
                               pl.ds(st * 128, 128)],
                    out_hbm.at[row, 2 * q + etp, st0 + st, :, :],
                    sem)
        return carry

    lax.fori_loop(0, D // 16, q_body, jnp.int32(0))
    for _k in range(32):
        pltpu.make_async_copy(
            trans_v.at[pl.ds(0, 8), pl.ds(0, 128)],
            out_hbm.at[row, 0, pl.ds(st0, 8), :, :].at[0],
            sem).wait()


def kernel(tokens, embed_table):
    tok = (tokens.astype(jnp.int32)
           .reshape(B, S // 128, 128).transpose(1, 0, 2))
    mesh = plsc.VectorSubcoreMesh(core_axis_name="c", subcore_axis_name="s")
    run_k = pl.kernel(
        _body,
        mesh=mesh,
        compiler_params=pltpu.CompilerParams(
            use_tc_tiling_on_sc=False, needs_layout_passes=False),
        out_type=jax.ShapeDtypeStruct((B, D // 8, S // 128, 8, 128),
                                      jnp.float32),
        scratch_types=[
            pltpu.VMEM((CHUNK // 128, 1, 128), jnp.int32),
            pltpu.VMEM((NGATHER, IDX_W), jnp.int32),
            pltpu.VMEM((CHUNK,), jnp.int32),
            pltpu.VMEM((CHUNK,), jnp.int32),
            pltpu.VMEM((CHUNK, D), jnp.float32),
            pltpu.VMEM((32, CHUNK + 1), jnp.float32),
            pltpu.VMEM((16,), jnp.int32),
            pltpu.VMEM((CPR, 16), jnp.int32),
            pltpu.VMEM_SHARED((NSUB, 16), jnp.int32),
            pltpu.SemaphoreType.DMA,
        ],
    )
    out5 = run_k(tok, embed_table)
    return out5.transpose(0, 2, 4, 1, 3).reshape(B, S, D)


# speedup vs baseline: 1.3222x; 1.2315x over previous
import jax
import jax.numpy as jnp
from jax import lax
from jax.experimental import pallas as pl
from jax.experimental.pallas import tpu as pltpu
from jax.experimental.pallas import tpu_sc as plsc

PAD = 1
B = 4
S = 8192
D = 64
NCORES = 2
NSUB = 16
NW = NCORES * NSUB
CHUNK = (B * S) // NW
CPR = S // CHUNK
ROWS_PER_CORE = B // NCORES
NVREG = CHUNK // 16
IDX_W = 128
NGATHER = CHUNK // IDX_W


def _body(tok_hbm, tab_hbm, out_hbm, tok_v, pos_v, aloc_v, mask_v, rows_v,
          trans_v, tot_v, grp_v, shared_tot, sem):
    c = lax.axis_index("c")
    s = lax.axis_index("s")
    row = c * ROWS_PER_CORE + s // CPR
    cir = s % CPR
    off = cir * CHUNK

    pltpu.sync_copy(tok_hbm.at[pl.ds(cir * (CHUNK // 128), CHUNK // 128),
                               pl.ds(row, 1), :], tok_v)

    pad_v = jnp.broadcast_to(jnp.int32(PAD), (16,))

    run = jnp.int32(0)
    for i in range(NVREG):
        t = tok_v[i // 8, 0, pl.ds((i % 8) * 16, 16)]
        m = (t != pad_v).astype(jnp.int32)
        cum = plsc.cumsum(m) + jnp.broadcast_to(run, (16,))
        aloc_v[pl.ds(i * 16, 16)] = cum * m
        mask_v[pl.ds(i * 16, 16)] = m
        run = run + jnp.sum(m)
    tot_v[...] = jnp.broadcast_to(run, (16,))
    pltpu.sync_copy(tot_v, shared_tot.at[s])
    plsc.subcore_barrier()

    grp_base = (s // CPR) * CPR
    pltpu.sync_copy(shared_tot.at[pl.ds(grp_base, CPR)], grp_v)
    offset = jnp.int32(0)
    for j in range(CPR):
        tj = jnp.max(grp_v[j, :])
        offset = offset + jnp.where(j < cir, tj, jnp.int32(0))

    off_v = jnp.broadcast_to(offset, (16,))
    for i in range(NVREG):
        a = aloc_v[pl.ds(i * 16, 16)]
        m = mask_v[pl.ds(i * 16, 16)]
        pos = a + off_v * m + pad_v
        pos_v[i // (IDX_W // 16), pl.ds((i % (IDX_W // 16)) * 16, 16)] = pos

    handles = [
        pltpu.async_copy(tab_hbm.at[pos_v.at[j]],
                         rows_v.at[pl.ds(j * IDX_W, IDX_W)], sem)
        for j in range(NGATHER)
    ]
    for h in handles:
        h.wait()

    row_iota = lax.iota(jnp.int32, 16)

    st0 = cir * (CHUNK // 128)
    nst = CHUNK // 128

    def q_body(q, carry):
        p = q % 2
        @pl.when(q >= 2)
        def _():
            for _k in range(16):
                pltpu.make_async_copy(
                    trans_v.at[pl.ds(0, 8), pl.ds(0, 128)],
                    out_hbm.at[row, 0, st0, :, :],
                    sem).wait()
        idx0_v = row_iota + jnp.broadcast_to(p * 16, (16,))

        def g_body(g, c2):
            s0 = g * 8
            vals = [rows_v[s0 + k, pl.ds(q * 16, 16)] for k in range(8)]
            for k in range(8):
                idx1_v = jnp.broadcast_to(s0 + k, (16,))
                plsc.store_scatter(trans_v, [idx0_v, idx1_v], vals[k])
            return c2

        lax.fori_loop(0, CHUNK // 8, g_body, jnp.int32(0))
        for etp in range(2):
            for st in range(nst):
                pltpu.async_copy(
                    trans_v.at[pl.ds(p * 16 + etp * 8, 8),
                               pl.ds(st * 128, 128)],
                    out_hbm.at[row, 2 * q + etp, st0 + st, :, :],
                    sem)
        return carry

    lax.fori_loop(0, D // 16, q_body, jnp.int32(0))
    for _k in range(32):
        pltpu.make_async_copy(
            trans_v.at[pl.ds(0, 8), pl.ds(0, 128)],
            out_hbm.at[row, 0, pl.ds(st0, 8), :, :].at[0],
            sem).wait()


def kernel(tokens, embed_table):
    tok = (tokens.astype(jnp.int32)
           .reshape(B, S // 128, 128).transpose(1, 0, 2))
    mesh = plsc.VectorSubcoreMesh(core_axis_name="c", subcore_axis_name="s")
    run_k = pl.kernel(
        _body,
        mesh=mesh,
        compiler_params=pltpu.CompilerParams(
            use_tc_tiling_on_sc=False, needs_layout_passes=False),
        out_type=jax.ShapeDtypeStruct((B, D // 8, S // 128, 8, 128),
                                      jnp.float32),
        scratch_types=[
            pltpu.VMEM((CHUNK // 128, 1, 128), jnp.int32),
            pltpu.VMEM((NGATHER, IDX_W), jnp.int32),
            pltpu.VMEM((CHUNK,), jnp.int32),
            pltpu.VMEM((CHUNK,), jnp.int32),
            pltpu.VMEM((CHUNK, D), jnp.float32),
            pltpu.VMEM((32, CHUNK + 1), jnp.float32),
            pltpu.VMEM((16,), jnp.int32),
            pltpu.VMEM((CPR, 16), jnp.int32),
            pltpu.VMEM_SHARED((NSUB, 16), jnp.int32),
            pltpu.SemaphoreType.DMA,
        ],
    )
    out5 = run_k(tok, embed_table)
    return out5.transpose(0, 2, 4, 1, 3).reshape(B, S, D)
